# baseline (speedup 1.0000x reference)
import functools

import jax
import jax.numpy as jnp
from jax import lax
from jax.experimental import pallas as pl
from jax.experimental.pallas import tpu as pltpu

_H = 4
_DH = 64


def _ln(x, g, b):
    mu = x.mean(-1, keepdims=True)
    v = ((x - mu) ** 2).mean(-1, keepdims=True)
    return (x - mu) / jnp.sqrt(v + 1e-5) * g + b


def _ffn_ln_body(relu_out, agg_ref, h_ref, w1_ref, b1_ref, w2_ref, b2_ref,
                 g_ref, b_ref, o_ref):
    z = jnp.maximum(agg_ref[...] @ w1_ref[...] + b1_ref[...], 0.0)
    z = z @ w2_ref[...] + b2_ref[...]
    y = _ln(h_ref[...] + z, g_ref[...], b_ref[...])
    if relu_out:
        y = jnp.maximum(y, 0.0)
    o_ref[...] = y


def _ffn_ln(agg, h, w1, b1, w2, b2, g, b, relu_out, bm=1000):
    m, d = agg.shape
    dh = w1.shape[1]
    b1 = b1.reshape(1, dh)
    b2 = b2.reshape(1, d)
    g = g.reshape(1, d)
    b = b.reshape(1, d)
    return pl.pallas_call(
        functools.partial(_ffn_ln_body, relu_out),
        grid=(m // bm,),
        in_specs=[
            pl.BlockSpec((bm, d), lambda i: (i, 0)),
            pl.BlockSpec((bm, d), lambda i: (i, 0)),
            pl.BlockSpec((d, dh), lambda i: (0, 0)),
            pl.BlockSpec((1, dh), lambda i: (0, 0)),
            pl.BlockSpec((dh, d), lambda i: (0, 0)),
            pl.BlockSpec((1, d), lambda i: (0, 0)),
            pl.BlockSpec((1, d), lambda i: (0, 0)),
            pl.BlockSpec((1, d), lambda i: (0, 0)),
        ],
        out_specs=pl.BlockSpec((bm, d), lambda i: (i, 0)),
        out_shape=jax.ShapeDtypeStruct((m, d), jnp.float32),
    )(agg, h, w1, b1, w2, b2, g, b)


def _pe_update_body(pe_ref, agg_ref, w_ref, o_ref):
    o_ref[...] = pe_ref[...] + jnp.maximum(agg_ref[...] @ w_ref[...], 0.0)


def _pe_update(pe, agg, w, bm=1000):
    m, d = pe.shape
    return pl.pallas_call(
        _pe_update_body,
        grid=(m // bm,),
        in_specs=[
            pl.BlockSpec((bm, d), lambda i: (i, 0)),
            pl.BlockSpec((bm, d), lambda i: (i, 0)),
            pl.BlockSpec((d, d), lambda i: (0, 0)),
        ],
        out_specs=pl.BlockSpec((bm, d), lambda i: (i, 0)),
        out_shape=jax.ShapeDtypeStruct((m, d), jnp.float32),
    )(pe, agg, w)


def _matmul_body(x_ref, w_ref, o_ref):
    o_ref[...] = x_ref[...] @ w_ref[...]


def _matmul(x, w, bm=1000):
    m, d = x.shape
    k = w.shape[1]
    return pl.pallas_call(
        _matmul_body,
        grid=(m // bm,),
        in_specs=[
            pl.BlockSpec((bm, d), lambda i: (i, 0)),
            pl.BlockSpec((d, k), lambda i: (0, 0)),
        ],
        out_specs=pl.BlockSpec((bm, k), lambda i: (i, 0)),
        out_shape=jax.ShapeDtypeStruct((m, k), jnp.float32),
    )(x, w)


def _proj_ln_body(x_ref, att_ref, wo_ref, g_ref, b_ref, o_ref):
    o_ref[...] = _ln(x_ref[...] + att_ref[...] @ wo_ref[...],
                     g_ref[...], b_ref[...])


def _proj_ln(x, att, wo, g, b, bm=1000):
    m, d = x.shape
    g = g.reshape(1, d)
    b = b.reshape(1, d)
    return pl.pallas_call(
        _proj_ln_body,
        grid=(m // bm,),
        in_specs=[
            pl.BlockSpec((bm, d), lambda i: (i, 0)),
            pl.BlockSpec((bm, d), lambda i: (i, 0)),
            pl.BlockSpec((d, d), lambda i: (0, 0)),
            pl.BlockSpec((1, d), lambda i: (0, 0)),
            pl.BlockSpec((1, d), lambda i: (0, 0)),
        ],
        out_specs=pl.BlockSpec((bm, d), lambda i: (i, 0)),
        out_shape=jax.ShapeDtypeStruct((m, d), jnp.float32),
    )(x, att, wo, g, b)


def kernel(x_ab, edge_index_ab, edge_attr_ab, pe_ab, batch_list, x_ba,
           edge_index_ba, edge_attr_ba, edge_map_ab, params):
    n, d = x_ab.shape
    e_ab = edge_attr_ab.shape[0]
    ng = 128
    layers = len(params["ab"])

    src_ab, dst_ab = edge_index_ab[0], edge_index_ab[1]
    src_ba, dst_ba = edge_index_ba[0], edge_index_ba[1]

    h_ab = x_ab
    e_attr = edge_attr_ab
    pe = pe_ab
    h_ba = x_ba

    for l in range(layers):
        last_act = (l != layers - 1)
        p = params["ab"][l]
        msg = h_ab[src_ab] + e_attr
        agg = jax.ops.segment_sum(msg, dst_ab, num_segments=n)
        h_new = _ffn_ln(agg, h_ab, p["W1"], p["b1"], p["W2"], p["b2"],
                        p["ln_g"], p["ln_b"], relu_out=last_act)
        pe_agg = jax.ops.segment_sum(pe[src_ab], dst_ab, num_segments=n)
        pe = _pe_update(pe, pe_agg, p["Wpe"])
        h_ab = h_new

        q = params["ba"][l]
        msg_b = h_ba[src_ba] + edge_attr_ba
        agg_b = jax.ops.segment_sum(msg_b, dst_ba, num_segments=e_ab)
        h_ba = _ffn_ln(agg_b, h_ba, q["W1"], q["b1"], q["W2"], q["b2"],
                       q["ln_g"], q["ln_b"], relu_out=last_act)
        e_attr = h_ba[edge_map_ab]

    node_repr = h_ab
    x = node_repr + _matmul(pe, params["tr_pe_in"])
    for t in params["tr"]:
        wqkv = jnp.concatenate([t["Wq"], t["Wk"], t["Wv"]], axis=1)
        qkv = _matmul(x, wqkv)
        qh = qkv[:, :d].reshape(n, _H, _DH)
        kh = qkv[:, d:2 * d].reshape(n, _H, _DH)
        vh = qkv[:, 2 * d:].reshape(n, _H, _DH)
        sc = (qh[dst_ab] * kh[src_ab]).sum(-1) / jnp.sqrt(jnp.float32(_DH))
        m = jax.ops.segment_max(sc, dst_ab, num_segments=n)
        ex = jnp.exp(sc - m[dst_ab])
        s = jax.ops.segment_sum(ex, dst_ab, num_segments=n)
        alpha = ex / (s[dst_ab] + 1e-9)
        att = jax.ops.segment_sum(alpha[:, :, None] * vh[src_ab], dst_ab,
                                  num_segments=n)
        x = _proj_ln(x, att.reshape(n, d), t["Wo"], t["ln1_g"], t["ln1_b"])
        x = _ffn_ln(x, x, t["ffW1"], t["ffb1"], t["ffW2"], t["ffb2"],
                    t["ln2_g"], t["ln2_b"], relu_out=False)

    node_feat = node_repr + x
    pe_lin = _matmul(pe, jnp.pad(params["pe_out_W"], ((0, 0), (0, 112)))
                     )[:, :16] + params["pe_out_b"]
    pe_repr = pe_lin / (jnp.linalg.norm(pe_lin, axis=0, keepdims=True) + 1e-9)
    sums = jax.ops.segment_sum(node_feat, batch_list, num_segments=ng)
    counts = jax.ops.segment_sum(jnp.ones((n,), jnp.float32), batch_list,
                                 num_segments=ng)
    graph_repr = sums / jnp.maximum(counts, 1.0)[:, None]
    return graph_repr, pe_repr, batch_list
